# Initial kernel scaffold; baseline (speedup 1.0000x reference)
import jax
import jax.numpy as jnp
from jax.experimental import pallas as pl


def kernel(x, xp, alpha, beta):
    raise NotImplementedError("write your pallas kernel here")


# speedup vs baseline: 2.2887x; 2.2887x over previous
import functools

import jax
import jax.numpy as jnp
from jax import lax
from jax.experimental import pallas as pl
from jax.experimental.pallas import tpu as pltpu
from jax.experimental.pallas import tpu_sc as plsc

BATCH = 16384
N_SPACE = 8
LANES = 16

_info = plsc.get_sparse_core_info()
_NC, _NS = _info.num_cores, _info.num_subcores
_NW = _NC * _NS
_B_PER_W = BATCH // _NW
_VREGS = _B_PER_W // LANES

_mesh = plsc.VectorSubcoreMesh(core_axis_name="c", subcore_axis_name="s")


@functools.partial(
    pl.kernel,
    mesh=_mesh,
    out_type=jax.ShapeDtypeStruct((BATCH,), jnp.float32),
    compiler_params=pltpu.CompilerParams(needs_layout_passes=False),
    scratch_types=[
        pltpu.VMEM((2 * _B_PER_W,), jnp.int32),
        pltpu.VMEM((2 * _B_PER_W,), jnp.int32),
        pltpu.VMEM((N_SPACE * N_SPACE,), jnp.float32),
        pltpu.VMEM((LANES,), jnp.float32),
        pltpu.VMEM((_B_PER_W,), jnp.float32),
    ],
)
def _sc_kernel(x_hbm, xp_hbm, alpha_hbm, beta_hbm, out_hbm,
               xv, xpv, av, bv, ov):
    wid = lax.axis_index("s") * _NC + lax.axis_index("c")
    base = wid * (2 * _B_PER_W)

    pltpu.sync_copy(x_hbm.at[pl.ds(base, 2 * _B_PER_W)], xv)
    pltpu.sync_copy(xp_hbm.at[pl.ds(base, 2 * _B_PER_W)], xpv)
    pltpu.sync_copy(alpha_hbm, av)
    pltpu.sync_copy(beta_hbm, bv)

    beta = bv[...]
    lane2 = lax.iota(jnp.int32, LANES) * 2

    def body(j, _):
        off = lane2 + j * (2 * LANES)
        x0 = plsc.load_gather(xv, [off])
        x1 = plsc.load_gather(xv, [off + 1])
        xp0 = plsc.load_gather(xpv, [off])
        xp1 = plsc.load_gather(xpv, [off + 1])
        al = plsc.load_gather(av, [xp1 * N_SPACE + x1])
        dt = jnp.abs(x0 - xp0).astype(jnp.float32)
        ov[pl.ds(j * LANES, LANES)] = al * beta * jnp.exp(-beta * dt)
        return _

    lax.fori_loop(0, _VREGS, body, None)
    pltpu.sync_copy(ov, out_hbm.at[pl.ds(wid * _B_PER_W, _B_PER_W)])


def kernel(x, xp, alpha, beta):
    x_flat = x.reshape(2 * BATCH)
    xp_flat = xp.reshape(2 * BATCH)
    alpha_flat = alpha.reshape(N_SPACE * N_SPACE)
    beta_b = jnp.broadcast_to(beta.astype(jnp.float32), (LANES,))
    return _sc_kernel(x_flat, xp_flat, alpha_flat, beta_b)


# speedup vs baseline: 2.8413x; 1.2414x over previous
import functools

import jax
import jax.numpy as jnp
from jax import lax
from jax.experimental import pallas as pl
from jax.experimental.pallas import tpu as pltpu
from jax.experimental.pallas import tpu_sc as plsc

BATCH = 16384
N_SPACE = 8
LANES = 16

_info = plsc.get_sparse_core_info()
_NC, _NS = _info.num_cores, _info.num_subcores
_NW = _NC * _NS
_B_PER_W = BATCH // _NW
_VREGS = _B_PER_W // LANES
_CH = 256
_NCHUNK = _B_PER_W // _CH

_mesh = plsc.VectorSubcoreMesh(core_axis_name="c", subcore_axis_name="s")


@functools.partial(
    pl.kernel,
    mesh=_mesh,
    out_type=jax.ShapeDtypeStruct((BATCH,), jnp.float32),
    compiler_params=pltpu.CompilerParams(
        needs_layout_passes=False,
    ),
    scratch_types=[
        pltpu.VMEM((_CH, 2), jnp.int32),
        pltpu.VMEM((_CH, 2), jnp.int32),
        pltpu.VMEM((N_SPACE, N_SPACE), jnp.float32),
        pltpu.VMEM((LANES,), jnp.float32),
        pltpu.VMEM((_B_PER_W,), jnp.float32),
        pltpu.SemaphoreType.DMA,
        pltpu.SemaphoreType.DMA,
        pltpu.SemaphoreType.DMA,
        pltpu.SemaphoreType.DMA,
    ],
)
def _sc_kernel(x_hbm, xp_hbm, alpha_hbm, beta_hbm, out_hbm,
               xv, xpv, av, bv, ov, sem0, sem1, sem2, sem3):
    wid = lax.axis_index("s") * _NC + lax.axis_index("c")
    base = wid * _B_PER_W

    c2 = pltpu.async_copy(alpha_hbm, av, sem2)
    c3 = pltpu.async_copy(beta_hbm, bv.at[pl.ds(0, 1)], sem3)
    c2.wait()
    c3.wait()

    lane = lax.iota(jnp.int32, LANES)
    zero = jnp.zeros((LANES,), jnp.int32)
    one = jnp.ones((LANES,), jnp.int32)
    beta = bv[...][0]

    for ch in range(_NCHUNK):
        row0 = base + ch * _CH
        c0 = pltpu.async_copy(x_hbm.at[pl.ds(row0, _CH), :], xv, sem0)
        c1 = pltpu.async_copy(xp_hbm.at[pl.ds(row0, _CH), :], xpv, sem1)
        c0.wait()
        c1.wait()
        for j in range(_CH // LANES):
            rows = lane + j * LANES
            x0 = plsc.load_gather(xv, [rows, zero])
            x1 = plsc.load_gather(xv, [rows, one])
            xp0 = plsc.load_gather(xpv, [rows, zero])
            xp1 = plsc.load_gather(xpv, [rows, one])
            al = plsc.load_gather(av, [xp1, x1])
            dt = jnp.abs(x0 - xp0).astype(jnp.float32)
            ov[pl.ds(ch * _CH + j * LANES, LANES)] = (
                al * beta * jnp.exp(-beta * dt))

    pltpu.sync_copy(ov, out_hbm.at[pl.ds(base, _B_PER_W)])


def kernel(x, xp, alpha, beta):
    return _sc_kernel(x, xp, alpha, beta)


# speedup vs baseline: 4.9985x; 1.7592x over previous
import functools

import jax
import jax.numpy as jnp
from jax import lax
from jax.experimental import pallas as pl
from jax.experimental.pallas import tpu as pltpu
from jax.experimental.pallas import tpu_sc as plsc

BATCH = 16384
N_SPACE = 8
LANES = 16

_info = plsc.get_sparse_core_info()
_NC, _NS = _info.num_cores, _info.num_subcores
_NW = _NC * _NS
_B_PER_W = BATCH // _NW
_VREGS = _B_PER_W // LANES

_mesh = plsc.VectorSubcoreMesh(core_axis_name="c", subcore_axis_name="s")


@functools.partial(
    pl.kernel,
    mesh=_mesh,
    out_type=jax.ShapeDtypeStruct((BATCH,), jnp.float32),
    compiler_params=pltpu.CompilerParams(needs_layout_passes=False),
    scratch_types=[
        pltpu.VMEM((_B_PER_W,), jnp.int32),
        pltpu.VMEM((_B_PER_W,), jnp.int32),
        pltpu.VMEM((_B_PER_W,), jnp.int32),
        pltpu.VMEM((_B_PER_W,), jnp.int32),
        pltpu.VMEM((N_SPACE, N_SPACE), jnp.float32),
        pltpu.VMEM((LANES,), jnp.float32),
        pltpu.VMEM((_B_PER_W,), jnp.float32),
        pltpu.SemaphoreType.DMA,
        pltpu.SemaphoreType.DMA,
        pltpu.SemaphoreType.DMA,
        pltpu.SemaphoreType.DMA,
        pltpu.SemaphoreType.DMA,
        pltpu.SemaphoreType.DMA,
    ],
)
def _sc_kernel(xt_hbm, xpt_hbm, alpha_hbm, beta_hbm, out_hbm,
               x0v, x1v, xp0v, xp1v, av, bv, ov,
               sem0, sem1, sem2, sem3, sem4, sem5):
    wid = lax.axis_index("s") * _NC + lax.axis_index("c")
    base = wid * _B_PER_W

    c0 = pltpu.async_copy(xt_hbm.at[0, pl.ds(base, _B_PER_W)], x0v, sem0)
    c1 = pltpu.async_copy(xt_hbm.at[1, pl.ds(base, _B_PER_W)], x1v, sem1)
    c2 = pltpu.async_copy(xpt_hbm.at[0, pl.ds(base, _B_PER_W)], xp0v, sem2)
    c3 = pltpu.async_copy(xpt_hbm.at[1, pl.ds(base, _B_PER_W)], xp1v, sem3)
    c4 = pltpu.async_copy(alpha_hbm, av, sem4)
    c5 = pltpu.async_copy(beta_hbm, bv.at[pl.ds(0, 1)], sem5)
    c0.wait()
    c1.wait()
    c2.wait()
    c3.wait()
    c4.wait()
    c5.wait()

    beta = bv[...][0]

    for j in range(_VREGS):
        sl = pl.ds(j * LANES, LANES)
        x0 = x0v[sl]
        x1 = x1v[sl]
        xp0 = xp0v[sl]
        xp1 = xp1v[sl]
        al = plsc.load_gather(av, [xp1, x1])
        dt = jnp.abs(x0 - xp0).astype(jnp.float32)
        ov[sl] = al * beta * jnp.exp(-beta * dt)

    pltpu.sync_copy(ov, out_hbm.at[pl.ds(base, _B_PER_W)])


def kernel(x, xp, alpha, beta):
    return _sc_kernel(x.T, xp.T, alpha, beta)


# speedup vs baseline: 5.0362x; 1.0075x over previous
import functools

import jax
import jax.numpy as jnp
from jax import lax
from jax.experimental import pallas as pl
from jax.experimental.pallas import tpu as pltpu
from jax.experimental.pallas import tpu_sc as plsc

BATCH = 16384
N_SPACE = 8
LANES = 16

_info = plsc.get_sparse_core_info()
_NC, _NS = _info.num_cores, _info.num_subcores
_NW = _NC * _NS
_B_PER_W = BATCH // _NW
_VREGS = _B_PER_W // LANES

_mesh = plsc.VectorSubcoreMesh(core_axis_name="c", subcore_axis_name="s")


@functools.partial(
    pl.kernel,
    mesh=_mesh,
    out_type=jax.ShapeDtypeStruct((BATCH,), jnp.float32),
    compiler_params=pltpu.CompilerParams(needs_layout_passes=False),
    scratch_types=[
        pltpu.VMEM((_B_PER_W,), jnp.int32),
        pltpu.VMEM((_B_PER_W,), jnp.int32),
        pltpu.VMEM((_B_PER_W,), jnp.int32),
        pltpu.VMEM((_B_PER_W,), jnp.int32),
        pltpu.VMEM((N_SPACE, N_SPACE), jnp.float32),
        pltpu.VMEM((LANES,), jnp.float32),
        pltpu.VMEM((_B_PER_W,), jnp.float32),
        pltpu.SemaphoreType.DMA,
        pltpu.SemaphoreType.DMA,
        pltpu.SemaphoreType.DMA,
        pltpu.SemaphoreType.DMA,
        pltpu.SemaphoreType.DMA,
        pltpu.SemaphoreType.DMA,
    ],
)
def _sc_kernel(xt_hbm, xpt_hbm, alpha_hbm, beta_hbm, out_hbm,
               x0v, x1v, xp0v, xp1v, av, bv, ov,
               sem0, sem1, sem2, sem3, sem4, sem5):
    wid = lax.axis_index("s") * _NC + lax.axis_index("c")
    base = wid * _B_PER_W

    c0 = pltpu.async_copy(xt_hbm.at[0, pl.ds(base, _B_PER_W)], x0v, sem0)
    c1 = pltpu.async_copy(xt_hbm.at[1, pl.ds(base, _B_PER_W)], x1v, sem1)
    c2 = pltpu.async_copy(xpt_hbm.at[0, pl.ds(base, _B_PER_W)], xp0v, sem2)
    c3 = pltpu.async_copy(xpt_hbm.at[1, pl.ds(base, _B_PER_W)], xp1v, sem3)
    c4 = pltpu.async_copy(alpha_hbm, av, sem4)
    c5 = pltpu.async_copy(beta_hbm, bv.at[pl.ds(0, 1)], sem5)
    c0.wait()
    c1.wait()
    c2.wait()
    c3.wait()
    c4.wait()
    c5.wait()

    beta = bv[...][0]

    @pl.loop(0, _VREGS, unroll=4)
    def _compute(j):
        sl = pl.ds(j * LANES, LANES)
        x0 = x0v[sl]
        x1 = x1v[sl]
        xp0 = xp0v[sl]
        xp1 = xp1v[sl]
        al = plsc.load_gather(av, [xp1, x1])
        dt = jnp.abs(x0 - xp0).astype(jnp.float32)
        ov[sl] = al * beta * jnp.exp(-beta * dt)

    pltpu.sync_copy(ov, out_hbm.at[pl.ds(base, _B_PER_W)])


def kernel(x, xp, alpha, beta):
    return _sc_kernel(x.T, xp.T, alpha, beta)
